# Initial kernel scaffold; baseline (speedup 1.0000x reference)
import functools

import jax
import jax.numpy as jnp
from jax import lax
from jax.experimental import pallas as pl
from jax.experimental.pallas import tpu as pltpu
from jax.experimental.pallas import tpu_sc as plsc

N = 10000
D = 128
E = 320000

NC = 2
NS = 16
NW = NC * NS
EPW = E // NW
CH = 80
NCHUNK = EPW // CH
RPT = N // NS

_mesh = plsc.VectorSubcoreMesh(core_axis_name="c", subcore_axis_name="s")


@functools.partial(
    pl.kernel,
    out_type=jax.ShapeDtypeStruct((NC, N, 16), jnp.float32),
    mesh=_mesh,
    scratch_types=[
        pltpu.VMEM((CH,), jnp.int32),
        pltpu.VMEM((CH, 16), jnp.float32),
        pltpu.VMEM_SHARED((N, 16), jnp.float32),
    ],
)
def _deg(dst_hbm, zeros_hbm, ones_hbm, out_hbm, idx_v, ones_v, acc):
    c = lax.axis_index("c")
    s = lax.axis_index("s")
    wid = c * NS + s
    pltpu.sync_copy(zeros_hbm.at[pl.ds(s * RPT, RPT)], acc.at[pl.ds(s * RPT, RPT)])
    pltpu.sync_copy(ones_hbm, ones_v)
    plsc.subcore_barrier()
    base = wid * EPW

    @pl.loop(0, NCHUNK)
    def _(k):
        pltpu.sync_copy(dst_hbm.at[pl.ds(base + k * CH, CH)], idx_v)
        pltpu.sync_copy(ones_v, acc.at[idx_v], add=True)

    plsc.subcore_barrier()
    pltpu.sync_copy(acc.at[pl.ds(s * RPT, RPT)], out_hbm.at[c].at[pl.ds(s * RPT, RPT)])


@functools.partial(
    pl.kernel,
    out_type=jax.ShapeDtypeStruct((NC, N, D), jnp.float32),
    mesh=_mesh,
    scratch_types=[
        pltpu.VMEM((CH,), jnp.int32),
        pltpu.VMEM((CH,), jnp.int32),
        pltpu.VMEM((CH, D), jnp.float32),
        pltpu.VMEM_SHARED((N, D), jnp.float32),
        pltpu.SemaphoreType.DMA,
    ],
)
def _agg(hp_hbm, src_hbm, dst_hbm, zeros_hbm, out_hbm, src_v, dst_v, rows_v, acc, sem):
    c = lax.axis_index("c")
    s = lax.axis_index("s")
    wid = c * NS + s
    pltpu.sync_copy(zeros_hbm.at[pl.ds(s * RPT, RPT)], acc.at[pl.ds(s * RPT, RPT)])
    plsc.subcore_barrier()
    base = wid * EPW

    @pl.loop(0, NCHUNK)
    def _(k):
        pltpu.sync_copy(src_hbm.at[pl.ds(base + k * CH, CH)], src_v)
        pltpu.sync_copy(dst_hbm.at[pl.ds(base + k * CH, CH)], dst_v)
        pltpu.async_copy(hp_hbm.at[src_v], rows_v, sem).wait()
        pltpu.sync_copy(rows_v, acc.at[dst_v], add=True)

    plsc.subcore_barrier()
    pltpu.sync_copy(acc.at[pl.ds(s * RPT, RPT)], out_hbm.at[c].at[pl.ds(s * RPT, RPT)])


R = 1000
G = N // R


def _dinv_of(degp_ref):
    d = degp_ref[0][:, 0:1] + degp_ref[1][:, 0:1] + 1.0
    return lax.rsqrt(jnp.maximum(d, 1.0))


def _pre_body(x_ref, w_ref, degp_ref, h_ref, hp_ref):
    dinv = _dinv_of(degp_ref)
    h = jnp.dot(x_ref[...], w_ref[...], preferred_element_type=jnp.float32)
    h_ref[...] = h
    hp_ref[...] = h * dinv


_pre = pl.pallas_call(
    _pre_body,
    grid=(G,),
    in_specs=[
        pl.BlockSpec((R, D), lambda i: (i, 0)),
        pl.BlockSpec((D, D), lambda i: (0, 0)),
        pl.BlockSpec((NC, R, 16), lambda i: (0, i, 0)),
    ],
    out_specs=[pl.BlockSpec((R, D), lambda i: (i, 0)),
               pl.BlockSpec((R, D), lambda i: (i, 0))],
    out_shape=[jax.ShapeDtypeStruct((N, D), jnp.float32),
               jax.ShapeDtypeStruct((N, D), jnp.float32)],
)


def _mid_body(agg_ref, h1_ref, degp_ref, b1_ref, w2_ref, h2_ref, hp2_ref):
    dinv = _dinv_of(degp_ref)
    a = agg_ref[0] + agg_ref[1]
    z = jnp.maximum((a + h1_ref[...] * dinv) * dinv + b1_ref[...], 0.0)
    h2 = jnp.dot(z, w2_ref[...], preferred_element_type=jnp.float32)
    h2_ref[...] = h2
    hp2_ref[...] = h2 * dinv


_mid = pl.pallas_call(
    _mid_body,
    grid=(G,),
    in_specs=[
        pl.BlockSpec((NC, R, D), lambda i: (0, i, 0)),
        pl.BlockSpec((R, D), lambda i: (i, 0)),
        pl.BlockSpec((NC, R, 16), lambda i: (0, i, 0)),
        pl.BlockSpec((1, D), lambda i: (0, 0)),
        pl.BlockSpec((D, D), lambda i: (0, 0)),
    ],
    out_specs=[pl.BlockSpec((R, D), lambda i: (i, 0)),
               pl.BlockSpec((R, D), lambda i: (i, 0))],
    out_shape=[jax.ShapeDtypeStruct((N, D), jnp.float32),
               jax.ShapeDtypeStruct((N, D), jnp.float32)],
)


def _post_body(agg_ref, h2_ref, degp_ref, b2_ref, out_ref):
    dinv = _dinv_of(degp_ref)
    a = agg_ref[0] + agg_ref[1]
    out_ref[...] = (a + h2_ref[...] * dinv) * dinv + b2_ref[...]


_post = pl.pallas_call(
    _post_body,
    grid=(G,),
    in_specs=[
        pl.BlockSpec((NC, R, D), lambda i: (0, i, 0)),
        pl.BlockSpec((R, D), lambda i: (i, 0)),
        pl.BlockSpec((NC, R, 16), lambda i: (0, i, 0)),
        pl.BlockSpec((1, D), lambda i: (0, 0)),
    ],
    out_specs=pl.BlockSpec((R, D), lambda i: (i, 0)),
    out_shape=jax.ShapeDtypeStruct((N, D), jnp.float32),
)


def kernel(x, edge_index, W1, b1, W2, b2):
    src = edge_index[0]
    dst = edge_index[1]
    zeros128 = jnp.zeros((N, D), jnp.float32)
    zeros16 = jnp.zeros((N, 16), jnp.float32)
    ones16 = jnp.ones((CH, 16), jnp.float32)

    degp = _deg(dst, zeros16, ones16)
    h1, hp1 = _pre(x, W1, degp)
    agg1 = _agg(hp1, src, dst, zeros128)
    h2, hp2 = _mid(agg1, h1, degp, b1.reshape(1, D), W2)
    agg2 = _agg(hp2, src, dst, zeros128)
    return _post(agg2, h2, degp, b2.reshape(1, D))


# speedup vs baseline: 13.9054x; 13.9054x over previous
import functools

import jax
import jax.numpy as jnp
from jax import lax
from jax.experimental import pallas as pl
from jax.experimental.pallas import tpu as pltpu
from jax.experimental.pallas import tpu_sc as plsc

N = 10000
D = 128
E = 320000

NC = 2
NS = 16
NW = NC * NS
EPW = E // NW
CH = 80
NCHUNK = EPW // CH
RPT = 624
REM = N - NS * RPT

_mesh = plsc.VectorSubcoreMesh(core_axis_name="c", subcore_axis_name="s")


N_PAD = 10240
DRPT = N_PAD // NS


@functools.partial(
    pl.kernel,
    out_type=jax.ShapeDtypeStruct((NC * N_PAD,), jnp.float32),
    mesh=_mesh,
    scratch_types=[
        pltpu.VMEM((EPW,), jnp.int32),
        pltpu.VMEM((N_PAD,), jnp.float32),
        pltpu.VMEM((DRPT,), jnp.float32),
        pltpu.VMEM((DRPT,), jnp.float32),
        pltpu.VMEM_SHARED((NS, N_PAD), jnp.float32),
    ],
    compiler_params=pltpu.CompilerParams(needs_layout_passes=False),
)
def _deg(dst_hbm, out_hbm, dstbuf, hist, accv, tmpv, stage):
    c = lax.axis_index("c")
    s = lax.axis_index("s")
    wid = c * NS + s
    pltpu.sync_copy(dst_hbm.at[pl.ds(wid * EPW, EPW)], dstbuf)
    zero16 = jnp.zeros((16,), jnp.float32)

    @pl.loop(0, N_PAD, step=16)
    def _(i):
        hist[pl.ds(i, 16)] = zero16

    one16 = jnp.ones((16,), jnp.float32)

    @pl.loop(0, EPW, step=16)
    def _(j):
        idx = dstbuf[pl.ds(j, 16)]
        plsc.addupdate_scatter(hist, [idx], one16)

    pltpu.sync_copy(hist, stage.at[s])
    plsc.subcore_barrier()
    row0 = s * DRPT
    pltpu.sync_copy(stage.at[0].at[pl.ds(row0, DRPT)], accv)

    @pl.loop(1, NS)
    def _(j):
        pltpu.sync_copy(stage.at[j].at[pl.ds(row0, DRPT)], tmpv)

        @pl.loop(0, DRPT, step=16)
        def _(i):
            accv[pl.ds(i, 16)] = accv[pl.ds(i, 16)] + tmpv[pl.ds(i, 16)]

    pltpu.sync_copy(accv, out_hbm.at[pl.ds(c * N_PAD + row0, DRPT)])


@functools.partial(
    pl.kernel,
    out_type=jax.ShapeDtypeStruct((NC, N, D), jnp.float32),
    mesh=_mesh,
    scratch_types=[
        pltpu.VMEM((CH,), jnp.int32),
        pltpu.VMEM((CH,), jnp.int32),
        pltpu.VMEM((CH, D), jnp.float32),
        pltpu.VMEM_SHARED((N, D), jnp.float32),
        pltpu.SemaphoreType.DMA,
    ],
)
def _agg(hp_hbm, src_hbm, dst_hbm, zeros_hbm, out_hbm, src_v, dst_v, rows_v, acc, sem):
    c = lax.axis_index("c")
    s = lax.axis_index("s")
    wid = c * NS + s
    pltpu.sync_copy(zeros_hbm.at[pl.ds(s * RPT, RPT)], acc.at[pl.ds(s * RPT, RPT)])

    @pl.when(s == NS - 1)
    def _():
        pltpu.sync_copy(zeros_hbm.at[pl.ds(NS * RPT, REM)], acc.at[pl.ds(NS * RPT, REM)])

    plsc.subcore_barrier()
    base = wid * EPW

    @pl.loop(0, NCHUNK)
    def _(k):
        pltpu.sync_copy(src_hbm.at[pl.ds(base + k * CH, CH)], src_v)
        pltpu.sync_copy(dst_hbm.at[pl.ds(base + k * CH, CH)], dst_v)
        pltpu.async_copy(hp_hbm.at[src_v], rows_v, sem).wait()
        pltpu.sync_copy(rows_v, acc.at[dst_v], add=True)

    plsc.subcore_barrier()
    pltpu.sync_copy(acc.at[pl.ds(s * RPT, RPT)], out_hbm.at[c].at[pl.ds(s * RPT, RPT)])

    @pl.when(s == NS - 1)
    def _():
        pltpu.sync_copy(acc.at[pl.ds(NS * RPT, REM)], out_hbm.at[c].at[pl.ds(NS * RPT, REM)])


R = 1000
G = N // R


def _dinv_of(d0_ref, d1_ref):
    d = d0_ref[...] + d1_ref[...] + 1.0
    return lax.rsqrt(jnp.maximum(d, 1.0))


_DEG_SPECS = [pl.BlockSpec((R, 1), lambda i: (i, 0)),
              pl.BlockSpec((R, 1), lambda i: (G + i, 0))]


def _pre_body(x_ref, w_ref, d0_ref, d1_ref, h_ref, hp_ref):
    dinv = _dinv_of(d0_ref, d1_ref)
    h = jnp.dot(x_ref[...], w_ref[...], preferred_element_type=jnp.float32)
    h_ref[...] = h
    hp_ref[...] = h * dinv


_pre = pl.pallas_call(
    _pre_body,
    grid=(G,),
    in_specs=[
        pl.BlockSpec((R, D), lambda i: (i, 0)),
        pl.BlockSpec((D, D), lambda i: (0, 0)),
        *_DEG_SPECS,
    ],
    out_specs=[pl.BlockSpec((R, D), lambda i: (i, 0)),
               pl.BlockSpec((R, D), lambda i: (i, 0))],
    out_shape=[jax.ShapeDtypeStruct((N, D), jnp.float32),
               jax.ShapeDtypeStruct((N, D), jnp.float32)],
)


def _mid_body(agg_ref, h1_ref, d0_ref, d1_ref, b1_ref, w2_ref, h2_ref, hp2_ref):
    dinv = _dinv_of(d0_ref, d1_ref)
    a = agg_ref[0] + agg_ref[1]
    z = jnp.maximum((a + h1_ref[...] * dinv) * dinv + b1_ref[...], 0.0)
    h2 = jnp.dot(z, w2_ref[...], preferred_element_type=jnp.float32)
    h2_ref[...] = h2
    hp2_ref[...] = h2 * dinv


_mid = pl.pallas_call(
    _mid_body,
    grid=(G,),
    in_specs=[
        pl.BlockSpec((NC, R, D), lambda i: (0, i, 0)),
        pl.BlockSpec((R, D), lambda i: (i, 0)),
        *_DEG_SPECS,
        pl.BlockSpec((1, D), lambda i: (0, 0)),
        pl.BlockSpec((D, D), lambda i: (0, 0)),
    ],
    out_specs=[pl.BlockSpec((R, D), lambda i: (i, 0)),
               pl.BlockSpec((R, D), lambda i: (i, 0))],
    out_shape=[jax.ShapeDtypeStruct((N, D), jnp.float32),
               jax.ShapeDtypeStruct((N, D), jnp.float32)],
)


def _post_body(agg_ref, h2_ref, d0_ref, d1_ref, b2_ref, out_ref):
    dinv = _dinv_of(d0_ref, d1_ref)
    a = agg_ref[0] + agg_ref[1]
    out_ref[...] = (a + h2_ref[...] * dinv) * dinv + b2_ref[...]


_post = pl.pallas_call(
    _post_body,
    grid=(G,),
    in_specs=[
        pl.BlockSpec((NC, R, D), lambda i: (0, i, 0)),
        pl.BlockSpec((R, D), lambda i: (i, 0)),
        *_DEG_SPECS,
        pl.BlockSpec((1, D), lambda i: (0, 0)),
    ],
    out_specs=pl.BlockSpec((R, D), lambda i: (i, 0)),
    out_shape=jax.ShapeDtypeStruct((N, D), jnp.float32),
)


def kernel(x, edge_index, W1, b1, W2, b2):
    src = edge_index[0]
    dst = edge_index[1]
    zeros128 = jnp.zeros((N, D), jnp.float32)

    degf = _deg(dst).reshape(NC, N_PAD)[:, :N].reshape(NC * N, 1)
    h1, hp1 = _pre(x, W1, degf, degf)
    agg1 = _agg(hp1, src, dst, zeros128)
    h2, hp2 = _mid(agg1, h1, degf, degf, b1.reshape(1, D), W2)
    agg2 = _agg(hp2, src, dst, zeros128)
    return _post(agg2, h2, degf, degf, b2.reshape(1, D))


# speedup vs baseline: 29.4129x; 2.1152x over previous
import functools

import jax
import jax.numpy as jnp
from jax import lax
from jax.experimental import pallas as pl
from jax.experimental.pallas import tpu as pltpu
from jax.experimental.pallas import tpu_sc as plsc

N = 10000
D = 128
E = 320000

NC = 2
NS = 16
NW = NC * NS
EPW = E // NW
CH = 80
NCHUNK = EPW // CH
RPT = 624
REM = N - NS * RPT

_mesh = plsc.VectorSubcoreMesh(core_axis_name="c", subcore_axis_name="s")


N_PAD = 10240
DRPT = N_PAD // NS


@functools.partial(
    pl.kernel,
    out_type=jax.ShapeDtypeStruct((NC * N_PAD,), jnp.float32),
    mesh=_mesh,
    scratch_types=[
        pltpu.VMEM((EPW,), jnp.int32),
        pltpu.VMEM((N_PAD,), jnp.float32),
        pltpu.VMEM((DRPT,), jnp.float32),
        pltpu.VMEM((DRPT,), jnp.float32),
        pltpu.VMEM_SHARED((NS, N_PAD), jnp.float32),
    ],
    compiler_params=pltpu.CompilerParams(needs_layout_passes=False),
)
def _deg(dst_hbm, out_hbm, dstbuf, hist, accv, tmpv, stage):
    c = lax.axis_index("c")
    s = lax.axis_index("s")
    wid = c * NS + s
    pltpu.sync_copy(dst_hbm.at[pl.ds(wid * EPW, EPW)], dstbuf)
    zero16 = jnp.zeros((16,), jnp.float32)

    @pl.loop(0, N_PAD, step=16)
    def _(i):
        hist[pl.ds(i, 16)] = zero16

    one16 = jnp.ones((16,), jnp.float32)

    @pl.loop(0, EPW, step=16)
    def _(j):
        idx = dstbuf[pl.ds(j, 16)]
        plsc.addupdate_scatter(hist, [idx], one16)

    pltpu.sync_copy(hist, stage.at[s])
    plsc.subcore_barrier()
    row0 = s * DRPT
    pltpu.sync_copy(stage.at[0].at[pl.ds(row0, DRPT)], accv)

    @pl.loop(1, NS)
    def _(j):
        pltpu.sync_copy(stage.at[j].at[pl.ds(row0, DRPT)], tmpv)

        @pl.loop(0, DRPT, step=16)
        def _(i):
            accv[pl.ds(i, 16)] = accv[pl.ds(i, 16)] + tmpv[pl.ds(i, 16)]

    pltpu.sync_copy(accv, out_hbm.at[pl.ds(c * N_PAD + row0, DRPT)])


@functools.partial(
    pl.kernel,
    out_type=jax.ShapeDtypeStruct((NC, N, D), jnp.float32),
    mesh=_mesh,
    scratch_types=[
        pltpu.VMEM((EPW,), jnp.int32),
        pltpu.VMEM((NCHUNK, CH), jnp.int32),
        pltpu.VMEM((CH, D), jnp.float32),
        pltpu.VMEM((CH, D), jnp.float32),
        pltpu.VMEM_SHARED((N, D), jnp.float32),
        pltpu.SemaphoreType.DMA,
        pltpu.SemaphoreType.DMA,
    ],
)
def _agg(hp_hbm, src_hbm, dst_hbm, zeros_hbm, out_hbm,
         sbuf, dbuf, rows_a, rows_b, acc, gsa, gsb):
    c = lax.axis_index("c")
    s = lax.axis_index("s")
    wid = c * NS + s
    pltpu.sync_copy(src_hbm.at[pl.ds(wid * EPW, EPW)], sbuf)
    pltpu.sync_copy(dst_hbm.at[wid], dbuf)
    pltpu.sync_copy(zeros_hbm.at[pl.ds(s * RPT, RPT)], acc.at[pl.ds(s * RPT, RPT)])

    @pl.when(s == NS - 1)
    def _():
        pltpu.sync_copy(zeros_hbm.at[pl.ds(NS * RPT, REM)], acc.at[pl.ds(NS * RPT, REM)])

    plsc.subcore_barrier()

    pltpu.async_copy(hp_hbm.at[sbuf.at[pl.ds(0, CH)]], rows_a, gsa)

    @pl.loop(0, NCHUNK - 1, step=2)
    def _(k):
        pltpu.async_copy(hp_hbm.at[sbuf.at[pl.ds((k + 1) * CH, CH)]], rows_b, gsb)
        pltpu.make_async_copy(hp_hbm.at[sbuf.at[pl.ds(k * CH, CH)]], rows_a, gsa).wait()
        pltpu.sync_copy(rows_a, acc.at[dbuf.at[k]], add=True)

        @pl.when(k + 2 < NCHUNK)
        def _():
            pltpu.async_copy(hp_hbm.at[sbuf.at[pl.ds((k + 2) * CH, CH)]], rows_a, gsa)

        pltpu.make_async_copy(hp_hbm.at[sbuf.at[pl.ds((k + 1) * CH, CH)]], rows_b, gsb).wait()
        pltpu.sync_copy(rows_b, acc.at[dbuf.at[k + 1]], add=True)

    pltpu.make_async_copy(hp_hbm.at[sbuf.at[pl.ds((NCHUNK - 1) * CH, CH)]], rows_a, gsa).wait()
    pltpu.sync_copy(rows_a, acc.at[dbuf.at[NCHUNK - 1]], add=True)

    plsc.subcore_barrier()
    pltpu.sync_copy(acc.at[pl.ds(s * RPT, RPT)], out_hbm.at[c].at[pl.ds(s * RPT, RPT)])

    @pl.when(s == NS - 1)
    def _():
        pltpu.sync_copy(acc.at[pl.ds(NS * RPT, REM)], out_hbm.at[c].at[pl.ds(NS * RPT, REM)])


R = 1000
G = N // R


def _dinv_of(d0_ref, d1_ref):
    d = d0_ref[...] + d1_ref[...] + 1.0
    return lax.rsqrt(jnp.maximum(d, 1.0))


_DEG_SPECS = [pl.BlockSpec((R, 1), lambda i: (i, 0)),
              pl.BlockSpec((R, 1), lambda i: (G + i, 0))]


def _pre_body(x_ref, w_ref, d0_ref, d1_ref, h_ref, hp_ref):
    dinv = _dinv_of(d0_ref, d1_ref)
    h = jnp.dot(x_ref[...], w_ref[...], preferred_element_type=jnp.float32)
    h_ref[...] = h
    hp_ref[...] = h * dinv


_pre = pl.pallas_call(
    _pre_body,
    grid=(G,),
    in_specs=[
        pl.BlockSpec((R, D), lambda i: (i, 0)),
        pl.BlockSpec((D, D), lambda i: (0, 0)),
        *_DEG_SPECS,
    ],
    out_specs=[pl.BlockSpec((R, D), lambda i: (i, 0)),
               pl.BlockSpec((R, D), lambda i: (i, 0))],
    out_shape=[jax.ShapeDtypeStruct((N, D), jnp.float32),
               jax.ShapeDtypeStruct((N, D), jnp.float32)],
)


def _mid_body(agg_ref, h1_ref, d0_ref, d1_ref, b1_ref, w2_ref, h2_ref, hp2_ref):
    dinv = _dinv_of(d0_ref, d1_ref)
    a = agg_ref[0] + agg_ref[1]
    z = jnp.maximum((a + h1_ref[...] * dinv) * dinv + b1_ref[...], 0.0)
    h2 = jnp.dot(z, w2_ref[...], preferred_element_type=jnp.float32)
    h2_ref[...] = h2
    hp2_ref[...] = h2 * dinv


_mid = pl.pallas_call(
    _mid_body,
    grid=(G,),
    in_specs=[
        pl.BlockSpec((NC, R, D), lambda i: (0, i, 0)),
        pl.BlockSpec((R, D), lambda i: (i, 0)),
        *_DEG_SPECS,
        pl.BlockSpec((1, D), lambda i: (0, 0)),
        pl.BlockSpec((D, D), lambda i: (0, 0)),
    ],
    out_specs=[pl.BlockSpec((R, D), lambda i: (i, 0)),
               pl.BlockSpec((R, D), lambda i: (i, 0))],
    out_shape=[jax.ShapeDtypeStruct((N, D), jnp.float32),
               jax.ShapeDtypeStruct((N, D), jnp.float32)],
)


def _post_body(agg_ref, h2_ref, d0_ref, d1_ref, b2_ref, out_ref):
    dinv = _dinv_of(d0_ref, d1_ref)
    a = agg_ref[0] + agg_ref[1]
    out_ref[...] = (a + h2_ref[...] * dinv) * dinv + b2_ref[...]


_post = pl.pallas_call(
    _post_body,
    grid=(G,),
    in_specs=[
        pl.BlockSpec((NC, R, D), lambda i: (0, i, 0)),
        pl.BlockSpec((R, D), lambda i: (i, 0)),
        *_DEG_SPECS,
        pl.BlockSpec((1, D), lambda i: (0, 0)),
    ],
    out_specs=pl.BlockSpec((R, D), lambda i: (i, 0)),
    out_shape=jax.ShapeDtypeStruct((N, D), jnp.float32),
)


def kernel(x, edge_index, W1, b1, W2, b2):
    src = edge_index[0]
    dst = edge_index[1]
    dst3 = dst.reshape(NW, NCHUNK, CH)
    zeros128 = jnp.zeros((N, D), jnp.float32)

    degf = _deg(dst).reshape(NC, N_PAD)[:, :N].reshape(NC * N, 1)
    h1, hp1 = _pre(x, W1, degf, degf)
    agg1 = _agg(hp1, src, dst3, zeros128)
    h2, hp2 = _mid(agg1, h1, degf, degf, b1.reshape(1, D), W2)
    agg2 = _agg(hp2, src, dst3, zeros128)
    return _post(agg2, h2, degf, degf, b2.reshape(1, D))


# speedup vs baseline: 29.7684x; 1.0121x over previous
import functools

import jax
import jax.numpy as jnp
from jax import lax
from jax.experimental import pallas as pl
from jax.experimental.pallas import tpu as pltpu
from jax.experimental.pallas import tpu_sc as plsc

N = 10000
D = 128
E = 320000

NC = 2
NS = 16
NW = NC * NS
EPW = E // NW
CH = 80
NCHUNK = EPW // CH
RPT = 624
REM = N - NS * RPT

_mesh = plsc.VectorSubcoreMesh(core_axis_name="c", subcore_axis_name="s")


N_PAD = 10240
DRPT = N_PAD // NS


@functools.partial(
    pl.kernel,
    out_type=jax.ShapeDtypeStruct((NC * N,), jnp.float32),
    mesh=_mesh,
    scratch_types=[
        pltpu.VMEM((EPW,), jnp.int32),
        pltpu.VMEM((N_PAD,), jnp.float32),
        pltpu.VMEM((DRPT,), jnp.float32),
        pltpu.VMEM((DRPT,), jnp.float32),
        pltpu.VMEM_SHARED((NS, N_PAD), jnp.float32),
    ],
    compiler_params=pltpu.CompilerParams(needs_layout_passes=False),
)
def _deg(dst_hbm, out_hbm, dstbuf, hist, accv, tmpv, stage):
    c = lax.axis_index("c")
    s = lax.axis_index("s")
    wid = c * NS + s
    pltpu.sync_copy(dst_hbm.at[pl.ds(wid * EPW, EPW)], dstbuf)
    zero16 = jnp.zeros((16,), jnp.float32)

    @pl.loop(0, N_PAD, step=16)
    def _(i):
        hist[pl.ds(i, 16)] = zero16

    one16 = jnp.ones((16,), jnp.float32)

    @pl.loop(0, EPW, step=16)
    def _(j):
        idx = dstbuf[pl.ds(j, 16)]
        plsc.addupdate_scatter(hist, [idx], one16)

    pltpu.sync_copy(hist, stage.at[s])
    plsc.subcore_barrier()
    row0 = s * DRPT
    pltpu.sync_copy(stage.at[0].at[pl.ds(row0, DRPT)], accv)

    @pl.loop(1, NS)
    def _(j):
        pltpu.sync_copy(stage.at[j].at[pl.ds(row0, DRPT)], tmpv)

        @pl.loop(0, DRPT, step=16)
        def _(i):
            accv[pl.ds(i, 16)] = accv[pl.ds(i, 16)] + tmpv[pl.ds(i, 16)]

    @pl.when(s < NS - 1)
    def _():
        pltpu.sync_copy(accv, out_hbm.at[pl.ds(c * N + row0, DRPT)])

    @pl.when(s == NS - 1)
    def _():
        pltpu.sync_copy(accv.at[pl.ds(0, N - (NS - 1) * DRPT)],
                        out_hbm.at[pl.ds(c * N + row0, N - (NS - 1) * DRPT)])


@functools.partial(
    pl.kernel,
    out_type=jax.ShapeDtypeStruct((NC, N, D), jnp.float32),
    mesh=_mesh,
    scratch_types=[
        pltpu.VMEM((EPW,), jnp.int32),
        pltpu.VMEM((NCHUNK, CH), jnp.int32),
        pltpu.VMEM((CH, D), jnp.float32),
        pltpu.VMEM((CH, D), jnp.float32),
        pltpu.VMEM_SHARED((N, D), jnp.float32),
        pltpu.SemaphoreType.DMA,
        pltpu.SemaphoreType.DMA,
    ],
)
def _agg(hp_hbm, src_hbm, dst_hbm, zeros_hbm, out_hbm,
         sbuf, dbuf, rows_a, rows_b, acc, gsa, gsb):
    c = lax.axis_index("c")
    s = lax.axis_index("s")
    wid = c * NS + s
    pltpu.sync_copy(src_hbm.at[pl.ds(wid * EPW, EPW)], sbuf)
    pltpu.sync_copy(dst_hbm.at[wid], dbuf)
    pltpu.sync_copy(zeros_hbm.at[pl.ds(s * RPT, RPT)], acc.at[pl.ds(s * RPT, RPT)])

    @pl.when(s == NS - 1)
    def _():
        pltpu.sync_copy(zeros_hbm.at[pl.ds(NS * RPT, REM)], acc.at[pl.ds(NS * RPT, REM)])

    plsc.subcore_barrier()

    pltpu.async_copy(hp_hbm.at[sbuf.at[pl.ds(0, CH)]], rows_a, gsa)

    @pl.loop(0, NCHUNK - 1, step=2)
    def _(k):
        pltpu.async_copy(hp_hbm.at[sbuf.at[pl.ds((k + 1) * CH, CH)]], rows_b, gsb)
        pltpu.make_async_copy(hp_hbm.at[sbuf.at[pl.ds(k * CH, CH)]], rows_a, gsa).wait()
        pltpu.sync_copy(rows_a, acc.at[dbuf.at[k]], add=True)

        @pl.when(k + 2 < NCHUNK)
        def _():
            pltpu.async_copy(hp_hbm.at[sbuf.at[pl.ds((k + 2) * CH, CH)]], rows_a, gsa)

        pltpu.make_async_copy(hp_hbm.at[sbuf.at[pl.ds((k + 1) * CH, CH)]], rows_b, gsb).wait()
        pltpu.sync_copy(rows_b, acc.at[dbuf.at[k + 1]], add=True)

    pltpu.make_async_copy(hp_hbm.at[sbuf.at[pl.ds((NCHUNK - 1) * CH, CH)]], rows_a, gsa).wait()
    pltpu.sync_copy(rows_a, acc.at[dbuf.at[NCHUNK - 1]], add=True)

    plsc.subcore_barrier()
    pltpu.sync_copy(acc.at[pl.ds(s * RPT, RPT)], out_hbm.at[c].at[pl.ds(s * RPT, RPT)])

    @pl.when(s == NS - 1)
    def _():
        pltpu.sync_copy(acc.at[pl.ds(NS * RPT, REM)], out_hbm.at[c].at[pl.ds(NS * RPT, REM)])


R = 1000
G = N // R


def _dinv_of(d0_ref, d1_ref):
    d = d0_ref[...] + d1_ref[...] + 1.0
    return lax.rsqrt(jnp.maximum(d, 1.0))


_DEG_SPECS = [pl.BlockSpec((R, 1), lambda i: (i, 0)),
              pl.BlockSpec((R, 1), lambda i: (G + i, 0))]


def _pre_body(x_ref, w_ref, d0_ref, d1_ref, hp_ref):
    dinv = _dinv_of(d0_ref, d1_ref)
    h = jnp.dot(x_ref[...], w_ref[...], preferred_element_type=jnp.float32)
    hp_ref[...] = h * dinv


_pre = pl.pallas_call(
    _pre_body,
    grid=(G,),
    in_specs=[
        pl.BlockSpec((R, D), lambda i: (i, 0)),
        pl.BlockSpec((D, D), lambda i: (0, 0)),
        *_DEG_SPECS,
    ],
    out_specs=pl.BlockSpec((R, D), lambda i: (i, 0)),
    out_shape=jax.ShapeDtypeStruct((N, D), jnp.float32),
)


def _mid_body(agg_ref, hp1_ref, d0_ref, d1_ref, b1_ref, w2_ref, hp2_ref):
    dinv = _dinv_of(d0_ref, d1_ref)
    a = agg_ref[0] + agg_ref[1]
    z = jnp.maximum((a + hp1_ref[...]) * dinv + b1_ref[...], 0.0)
    h2 = jnp.dot(z, w2_ref[...], preferred_element_type=jnp.float32)
    hp2_ref[...] = h2 * dinv


_mid = pl.pallas_call(
    _mid_body,
    grid=(G,),
    in_specs=[
        pl.BlockSpec((NC, R, D), lambda i: (0, i, 0)),
        pl.BlockSpec((R, D), lambda i: (i, 0)),
        *_DEG_SPECS,
        pl.BlockSpec((1, D), lambda i: (0, 0)),
        pl.BlockSpec((D, D), lambda i: (0, 0)),
    ],
    out_specs=pl.BlockSpec((R, D), lambda i: (i, 0)),
    out_shape=jax.ShapeDtypeStruct((N, D), jnp.float32),
)


def _post_body(agg_ref, hp2_ref, d0_ref, d1_ref, b2_ref, out_ref):
    dinv = _dinv_of(d0_ref, d1_ref)
    a = agg_ref[0] + agg_ref[1]
    out_ref[...] = (a + hp2_ref[...]) * dinv + b2_ref[...]


_post = pl.pallas_call(
    _post_body,
    grid=(G,),
    in_specs=[
        pl.BlockSpec((NC, R, D), lambda i: (0, i, 0)),
        pl.BlockSpec((R, D), lambda i: (i, 0)),
        *_DEG_SPECS,
        pl.BlockSpec((1, D), lambda i: (0, 0)),
    ],
    out_specs=pl.BlockSpec((R, D), lambda i: (i, 0)),
    out_shape=jax.ShapeDtypeStruct((N, D), jnp.float32),
)


def kernel(x, edge_index, W1, b1, W2, b2):
    src = edge_index[0]
    dst = edge_index[1]
    dst3 = dst.reshape(NW, NCHUNK, CH)
    zeros128 = jnp.zeros((N, D), jnp.float32)

    degf = _deg(dst).reshape(NC * N, 1)
    hp1 = _pre(x, W1, degf, degf)
    agg1 = _agg(hp1, src, dst3, zeros128)
    hp2 = _mid(agg1, hp1, degf, degf, b1.reshape(1, D), W2)
    agg2 = _agg(hp2, src, dst3, zeros128)
    return _post(agg2, hp2, degf, degf, b2.reshape(1, D))


# speedup vs baseline: 33.9246x; 1.1396x over previous
import functools

import jax
import jax.numpy as jnp
from jax import lax
from jax.experimental import pallas as pl
from jax.experimental.pallas import tpu as pltpu
from jax.experimental.pallas import tpu_sc as plsc

N = 10000
D = 128
E = 320000

NC = 2
NS = 16
NW = NC * NS
EPW = E // NW
CH = 80
NCHUNK = EPW // CH
RPT = 624
REM = N - NS * RPT

_mesh = plsc.VectorSubcoreMesh(core_axis_name="c", subcore_axis_name="s")


N_PAD = 10240
DRPT = N_PAD // NS


@functools.partial(
    pl.kernel,
    out_type=jax.ShapeDtypeStruct((NC * N,), jnp.float32),
    mesh=_mesh,
    scratch_types=[
        pltpu.VMEM((EPW,), jnp.int32),
        pltpu.VMEM((N_PAD,), jnp.float32),
        pltpu.VMEM((DRPT,), jnp.float32),
        pltpu.VMEM((DRPT,), jnp.float32),
        pltpu.VMEM_SHARED((NS, N_PAD), jnp.float32),
    ],
    compiler_params=pltpu.CompilerParams(needs_layout_passes=False),
)
def _deg(dst_hbm, out_hbm, dstbuf, hist, accv, tmpv, stage):
    c = lax.axis_index("c")
    s = lax.axis_index("s")
    wid = c * NS + s
    pltpu.sync_copy(dst_hbm.at[pl.ds(wid * EPW, EPW)], dstbuf)
    zero16 = jnp.zeros((16,), jnp.float32)

    @pl.loop(0, N_PAD, step=16)
    def _(i):
        hist[pl.ds(i, 16)] = zero16

    one16 = jnp.ones((16,), jnp.float32)

    @pl.loop(0, EPW, step=16)
    def _(j):
        idx = dstbuf[pl.ds(j, 16)]
        plsc.addupdate_scatter(hist, [idx], one16)

    pltpu.sync_copy(hist, stage.at[s])
    plsc.subcore_barrier()
    row0 = s * DRPT
    pltpu.sync_copy(stage.at[0].at[pl.ds(row0, DRPT)], accv)

    @pl.loop(1, NS)
    def _(j):
        pltpu.sync_copy(stage.at[j].at[pl.ds(row0, DRPT)], tmpv)

        @pl.loop(0, DRPT, step=16)
        def _(i):
            accv[pl.ds(i, 16)] = accv[pl.ds(i, 16)] + tmpv[pl.ds(i, 16)]

    @pl.when(s < NS - 1)
    def _():
        pltpu.sync_copy(accv, out_hbm.at[pl.ds(c * N + row0, DRPT)])

    @pl.when(s == NS - 1)
    def _():
        pltpu.sync_copy(accv.at[pl.ds(0, N - (NS - 1) * DRPT)],
                        out_hbm.at[pl.ds(c * N + row0, N - (NS - 1) * DRPT)])


@functools.partial(
    pl.kernel,
    out_type=jax.ShapeDtypeStruct((NC, N, D), jnp.float32),
    mesh=_mesh,
    scratch_types=[
        pltpu.VMEM((NCHUNK, CH), jnp.int32),
        [pltpu.VMEM((CH,), jnp.int32) for _ in range(3)],
        [pltpu.VMEM((CH, D), jnp.float32) for _ in range(3)],
        pltpu.VMEM_SHARED((N, D), jnp.float32),
        [pltpu.SemaphoreType.DMA for _ in range(3)],
        [pltpu.SemaphoreType.DMA for _ in range(3)],
    ],
)
def _agg(hp_hbm, src_hbm, dst_hbm, zeros_hbm, out_hbm,
         dbuf, ib, rows, acc, gs, ss):
    c = lax.axis_index("c")
    s = lax.axis_index("s")
    wid = c * NS + s
    pltpu.sync_copy(dst_hbm.at[wid], dbuf)
    pltpu.sync_copy(zeros_hbm.at[pl.ds(s * RPT, RPT)], acc.at[pl.ds(s * RPT, RPT)])

    @pl.when(s == NS - 1)
    def _():
        pltpu.sync_copy(zeros_hbm.at[pl.ds(NS * RPT, REM)], acc.at[pl.ds(NS * RPT, REM)])

    plsc.subcore_barrier()
    base = wid * EPW

    for b in range(3):
        pltpu.sync_copy(src_hbm.at[pl.ds(base + b * CH, CH)], ib[b])
        pltpu.async_copy(hp_hbm.at[ib[b]], rows[b], gs[b])

    @pl.loop(0, NCHUNK - 4, step=3)
    def _(k):
        for b in range(3):
            kk = k + b
            pltpu.make_async_copy(hp_hbm.at[ib[b]], rows[b], gs[b]).wait()
            pltpu.async_copy(rows[b], acc.at[dbuf.at[kk]], ss[b], add=True)
            nk = kk + 3

            @pl.when(nk < NCHUNK)
            def _():
                pltpu.sync_copy(src_hbm.at[pl.ds(base + nk * CH, CH)], ib[b])

            pltpu.make_async_copy(rows[b], acc.at[dbuf.at[kk]], ss[b]).wait()

            @pl.when(nk < NCHUNK)
            def _():
                pltpu.async_copy(hp_hbm.at[ib[b]], rows[b], gs[b])

    for b, kk in ((0, NCHUNK - 2), (1, NCHUNK - 1)):
        pltpu.make_async_copy(hp_hbm.at[ib[b]], rows[b], gs[b]).wait()
        pltpu.sync_copy(rows[b], acc.at[dbuf.at[kk]], add=True)

    plsc.subcore_barrier()
    pltpu.sync_copy(acc.at[pl.ds(s * RPT, RPT)], out_hbm.at[c].at[pl.ds(s * RPT, RPT)])

    @pl.when(s == NS - 1)
    def _():
        pltpu.sync_copy(acc.at[pl.ds(NS * RPT, REM)], out_hbm.at[c].at[pl.ds(NS * RPT, REM)])


R = 1000
G = N // R


def _dinv_of(d0_ref, d1_ref):
    d = d0_ref[...] + d1_ref[...] + 1.0
    return lax.rsqrt(jnp.maximum(d, 1.0))


_DEG_SPECS = [pl.BlockSpec((R, 1), lambda i: (i, 0)),
              pl.BlockSpec((R, 1), lambda i: (G + i, 0))]


def _pre_body(x_ref, w_ref, d0_ref, d1_ref, hp_ref):
    dinv = _dinv_of(d0_ref, d1_ref)
    h = jnp.dot(x_ref[...], w_ref[...], preferred_element_type=jnp.float32)
    hp_ref[...] = h * dinv


_pre = pl.pallas_call(
    _pre_body,
    grid=(G,),
    in_specs=[
        pl.BlockSpec((R, D), lambda i: (i, 0)),
        pl.BlockSpec((D, D), lambda i: (0, 0)),
        *_DEG_SPECS,
    ],
    out_specs=pl.BlockSpec((R, D), lambda i: (i, 0)),
    out_shape=jax.ShapeDtypeStruct((N, D), jnp.float32),
)


def _mid_body(agg_ref, hp1_ref, d0_ref, d1_ref, b1_ref, w2_ref, hp2_ref):
    dinv = _dinv_of(d0_ref, d1_ref)
    a = agg_ref[0] + agg_ref[1]
    z = jnp.maximum((a + hp1_ref[...]) * dinv + b1_ref[...], 0.0)
    h2 = jnp.dot(z, w2_ref[...], preferred_element_type=jnp.float32)
    hp2_ref[...] = h2 * dinv


_mid = pl.pallas_call(
    _mid_body,
    grid=(G,),
    in_specs=[
        pl.BlockSpec((NC, R, D), lambda i: (0, i, 0)),
        pl.BlockSpec((R, D), lambda i: (i, 0)),
        *_DEG_SPECS,
        pl.BlockSpec((1, D), lambda i: (0, 0)),
        pl.BlockSpec((D, D), lambda i: (0, 0)),
    ],
    out_specs=pl.BlockSpec((R, D), lambda i: (i, 0)),
    out_shape=jax.ShapeDtypeStruct((N, D), jnp.float32),
)


def _post_body(agg_ref, hp2_ref, d0_ref, d1_ref, b2_ref, out_ref):
    dinv = _dinv_of(d0_ref, d1_ref)
    a = agg_ref[0] + agg_ref[1]
    out_ref[...] = (a + hp2_ref[...]) * dinv + b2_ref[...]


_post = pl.pallas_call(
    _post_body,
    grid=(G,),
    in_specs=[
        pl.BlockSpec((NC, R, D), lambda i: (0, i, 0)),
        pl.BlockSpec((R, D), lambda i: (i, 0)),
        *_DEG_SPECS,
        pl.BlockSpec((1, D), lambda i: (0, 0)),
    ],
    out_specs=pl.BlockSpec((R, D), lambda i: (i, 0)),
    out_shape=jax.ShapeDtypeStruct((N, D), jnp.float32),
)


def kernel(x, edge_index, W1, b1, W2, b2):
    src = edge_index[0]
    dst = edge_index[1]
    dst3 = dst.reshape(NW, NCHUNK, CH)
    zeros128 = jnp.zeros((N, D), jnp.float32)

    degf = _deg(dst).reshape(NC * N, 1)
    hp1 = _pre(x, W1, degf, degf)
    agg1 = _agg(hp1, src, dst3, zeros128)
    hp2 = _mid(agg1, hp1, degf, degf, b1.reshape(1, D), W2)
    agg2 = _agg(hp2, src, dst3, zeros128)
    return _post(agg2, hp2, degf, degf, b2.reshape(1, D))


# speedup vs baseline: 34.7008x; 1.0229x over previous
import functools

import jax
import jax.numpy as jnp
from jax import lax
from jax.experimental import pallas as pl
from jax.experimental.pallas import tpu as pltpu
from jax.experimental.pallas import tpu_sc as plsc

N = 10000
D = 128
E = 320000

NC = 2
NS = 16
NW = NC * NS
EPW = E // NW
CH = 80
NCHUNK = EPW // CH
RPT = 624
REM = N - NS * RPT

_mesh = plsc.VectorSubcoreMesh(core_axis_name="c", subcore_axis_name="s")


N_PAD = 10240
DRPT = N_PAD // NS


@functools.partial(
    pl.kernel,
    out_type=jax.ShapeDtypeStruct((NC * N,), jnp.float32),
    mesh=_mesh,
    scratch_types=[
        pltpu.VMEM((NCHUNK, CH), jnp.int32),
        pltpu.VMEM((N_PAD,), jnp.float32),
        pltpu.VMEM((DRPT,), jnp.float32),
        pltpu.VMEM((DRPT,), jnp.float32),
        pltpu.VMEM_SHARED((NS, N_PAD), jnp.float32),
    ],
    compiler_params=pltpu.CompilerParams(needs_layout_passes=False),
)
def _deg(dst_hbm, out_hbm, dstbuf, hist, accv, tmpv, stage):
    c = lax.axis_index("c")
    s = lax.axis_index("s")
    wid = c * NS + s
    pltpu.sync_copy(dst_hbm.at[wid], dstbuf)
    zero16 = jnp.zeros((16,), jnp.float32)

    @pl.loop(0, N_PAD, step=16)
    def _(i):
        hist[pl.ds(i, 16)] = zero16

    one16 = jnp.ones((16,), jnp.float32)

    @pl.loop(0, NCHUNK)
    def _(k):
        @pl.loop(0, CH, step=16)
        def _(j):
            idx = dstbuf[k, pl.ds(j, 16)]
            plsc.addupdate_scatter(hist, [idx], one16)

    pltpu.sync_copy(hist, stage.at[s])
    plsc.subcore_barrier()
    row0 = s * DRPT
    pltpu.sync_copy(stage.at[0].at[pl.ds(row0, DRPT)], accv)

    @pl.loop(1, NS)
    def _(j):
        pltpu.sync_copy(stage.at[j].at[pl.ds(row0, DRPT)], tmpv)

        @pl.loop(0, DRPT, step=16)
        def _(i):
            accv[pl.ds(i, 16)] = accv[pl.ds(i, 16)] + tmpv[pl.ds(i, 16)]

    @pl.when(s < NS - 1)
    def _():
        pltpu.sync_copy(accv, out_hbm.at[pl.ds(c * N + row0, DRPT)])

    @pl.when(s == NS - 1)
    def _():
        pltpu.sync_copy(accv.at[pl.ds(0, N - (NS - 1) * DRPT)],
                        out_hbm.at[pl.ds(c * N + row0, N - (NS - 1) * DRPT)])


@functools.partial(
    pl.kernel,
    out_type=jax.ShapeDtypeStruct((NC, N, D), jnp.float32),
    mesh=_mesh,
    scratch_types=[
        pltpu.VMEM((NCHUNK, CH), jnp.int32),
        [pltpu.VMEM((CH,), jnp.int32) for _ in range(3)],
        [pltpu.VMEM((CH, D), jnp.float32) for _ in range(3)],
        pltpu.VMEM_SHARED((N, D), jnp.float32),
        [pltpu.SemaphoreType.DMA for _ in range(3)],
        [pltpu.SemaphoreType.DMA for _ in range(3)],
    ],
)
def _agg(hp_hbm, src_hbm, dst_hbm, out_hbm,
         dbuf, ib, rows, acc, gs, ss):
    c = lax.axis_index("c")
    s = lax.axis_index("s")
    wid = c * NS + s
    pltpu.sync_copy(dst_hbm.at[wid], dbuf)

    zero16 = jnp.zeros((16,), jnp.float32)

    @pl.loop(0, CH)
    def _(r):
        @pl.loop(0, D, step=16)
        def _(l):
            rows[0][r, pl.ds(l, 16)] = zero16

    @pl.loop(0, 7)
    def _(i):
        pltpu.sync_copy(rows[0], acc.at[pl.ds(s * RPT + i * CH, CH)])

    pltpu.sync_copy(rows[0].at[pl.ds(0, RPT - 7 * CH)],
                    acc.at[pl.ds(s * RPT + 7 * CH, RPT - 7 * CH)])

    @pl.when(s == NS - 1)
    def _():
        pltpu.sync_copy(rows[0].at[pl.ds(0, REM)], acc.at[pl.ds(NS * RPT, REM)])

    plsc.subcore_barrier()
    base = wid * EPW

    for b in range(3):
        pltpu.sync_copy(src_hbm.at[pl.ds(base + b * CH, CH)], ib[b])
        pltpu.async_copy(hp_hbm.at[ib[b]], rows[b], gs[b])

    @pl.loop(0, NCHUNK - 4, step=3)
    def _(k):
        for b in range(3):
            kk = k + b
            pltpu.make_async_copy(hp_hbm.at[ib[b]], rows[b], gs[b]).wait()
            pltpu.async_copy(rows[b], acc.at[dbuf.at[kk]], ss[b], add=True)
            nk = kk + 3

            @pl.when(nk < NCHUNK)
            def _():
                pltpu.sync_copy(src_hbm.at[pl.ds(base + nk * CH, CH)], ib[b])

            pltpu.make_async_copy(rows[b], acc.at[dbuf.at[kk]], ss[b]).wait()

            @pl.when(nk < NCHUNK)
            def _():
                pltpu.async_copy(hp_hbm.at[ib[b]], rows[b], gs[b])

    for b, kk in ((0, NCHUNK - 2), (1, NCHUNK - 1)):
        pltpu.make_async_copy(hp_hbm.at[ib[b]], rows[b], gs[b]).wait()
        pltpu.sync_copy(rows[b], acc.at[dbuf.at[kk]], add=True)

    plsc.subcore_barrier()
    pltpu.sync_copy(acc.at[pl.ds(s * RPT, RPT)], out_hbm.at[c].at[pl.ds(s * RPT, RPT)])

    @pl.when(s == NS - 1)
    def _():
        pltpu.sync_copy(acc.at[pl.ds(NS * RPT, REM)], out_hbm.at[c].at[pl.ds(NS * RPT, REM)])


R = 1000
G = N // R


def _dinv_of(d0_ref, d1_ref):
    d = d0_ref[...] + d1_ref[...] + 1.0
    return lax.rsqrt(jnp.maximum(d, 1.0))


_DEG_SPECS = [pl.BlockSpec((R, 1), lambda i: (i, 0)),
              pl.BlockSpec((R, 1), lambda i: (G + i, 0))]


def _pre_body(x_ref, w_ref, d0_ref, d1_ref, hp_ref):
    dinv = _dinv_of(d0_ref, d1_ref)
    h = jnp.dot(x_ref[...], w_ref[...], preferred_element_type=jnp.float32)
    hp_ref[...] = h * dinv


_pre = pl.pallas_call(
    _pre_body,
    grid=(G,),
    in_specs=[
        pl.BlockSpec((R, D), lambda i: (i, 0)),
        pl.BlockSpec((D, D), lambda i: (0, 0)),
        *_DEG_SPECS,
    ],
    out_specs=pl.BlockSpec((R, D), lambda i: (i, 0)),
    out_shape=jax.ShapeDtypeStruct((N, D), jnp.float32),
)


def _mid_body(agg_ref, hp1_ref, d0_ref, d1_ref, b1_ref, w2_ref, hp2_ref):
    dinv = _dinv_of(d0_ref, d1_ref)
    a = agg_ref[0] + agg_ref[1]
    z = jnp.maximum((a + hp1_ref[...]) * dinv + b1_ref[...], 0.0)
    h2 = jnp.dot(z, w2_ref[...], preferred_element_type=jnp.float32)
    hp2_ref[...] = h2 * dinv


_mid = pl.pallas_call(
    _mid_body,
    grid=(G,),
    in_specs=[
        pl.BlockSpec((NC, R, D), lambda i: (0, i, 0)),
        pl.BlockSpec((R, D), lambda i: (i, 0)),
        *_DEG_SPECS,
        pl.BlockSpec((1, D), lambda i: (0, 0)),
        pl.BlockSpec((D, D), lambda i: (0, 0)),
    ],
    out_specs=pl.BlockSpec((R, D), lambda i: (i, 0)),
    out_shape=jax.ShapeDtypeStruct((N, D), jnp.float32),
)


def _post_body(agg_ref, hp2_ref, d0_ref, d1_ref, b2_ref, out_ref):
    dinv = _dinv_of(d0_ref, d1_ref)
    a = agg_ref[0] + agg_ref[1]
    out_ref[...] = (a + hp2_ref[...]) * dinv + b2_ref[...]


_post = pl.pallas_call(
    _post_body,
    grid=(G,),
    in_specs=[
        pl.BlockSpec((NC, R, D), lambda i: (0, i, 0)),
        pl.BlockSpec((R, D), lambda i: (i, 0)),
        *_DEG_SPECS,
        pl.BlockSpec((1, D), lambda i: (0, 0)),
    ],
    out_specs=pl.BlockSpec((R, D), lambda i: (i, 0)),
    out_shape=jax.ShapeDtypeStruct((N, D), jnp.float32),
)


def kernel(x, edge_index, W1, b1, W2, b2):
    src = edge_index[0]
    dst3 = edge_index[1].reshape(NW, NCHUNK, CH)

    degf = _deg(dst3).reshape(NC * N, 1)
    hp1 = _pre(x, W1, degf, degf)
    agg1 = _agg(hp1, src, dst3)
    hp2 = _mid(agg1, hp1, degf, degf, b1.reshape(1, D), W2)
    agg2 = _agg(hp2, src, dst3)
    return _post(agg2, hp2, degf, degf, b2.reshape(1, D))
